# baseline (speedup 1.0000x reference)
import functools

import jax
import jax.numpy as jnp
from jax import lax
from jax.experimental import pallas as pl
from jax.experimental.pallas import tpu as pltpu
from jax.experimental.pallas import tpu_sc as plsc

DIM = 10000
SIZE = 32
LEVELS = 256
BATCH = 64

NW = 32
CPW = 320
DPAD = NW * CPW
NC16 = CPW // 16


def _sc_encode(img_t, vt_r, xt_r, yt_r):
    mesh = plsc.VectorSubcoreMesh(core_axis_name="c", subcore_axis_name="s")

    @functools.partial(
        pl.kernel,
        mesh=mesh,
        out_type=jax.ShapeDtypeStruct((NW, BATCH, CPW), jnp.float32),
        scratch_types=[
            pltpu.VMEM((LEVELS, CPW), jnp.float32),
            pltpu.VMEM((SIZE, CPW), jnp.float32),
            pltpu.VMEM((CPW,), jnp.float32),
            pltpu.VMEM((BATCH, CPW), jnp.float32),
            pltpu.VMEM((SIZE, BATCH), jnp.int32),
            pltpu.SemaphoreType.DMA,
            pltpu.SemaphoreType.DMA,
        ],
        compiler_params=pltpu.CompilerParams(use_tc_tiling_on_sc=False),
    )
    def enc(img_hbm, vt_hbm, xt_hbm, yt_hbm, out_hbm,
            vt_s, xt_s, yt_row, acc_s, idx_s, sem1, sem2):
        w = lax.axis_index("s") * 2 + lax.axis_index("c")
        cp1 = pltpu.async_copy(vt_hbm.at[w], vt_s, sem1)
        cp2 = pltpu.async_copy(xt_hbm.at[w], xt_s, sem2)

        zero = jnp.zeros((16,), jnp.float32)

        @pl.loop(0, BATCH)
        def _(b):
            for c in range(NC16):
                acc_s[b, pl.ds(c * 16, 16)] = zero

        cp1.wait()
        cp2.wait()

        @pl.loop(0, SIZE)
        def _(y):
            pltpu.sync_copy(img_hbm.at[pl.ds(y * SIZE, SIZE)], idx_s)
            pltpu.sync_copy(yt_hbm.at[w, y], yt_row)

            @pl.loop(0, SIZE)
            def _(xcol):
                wvec = [xt_s[xcol, pl.ds(c * 16, 16)] * yt_row[pl.ds(c * 16, 16)]
                        for c in range(NC16)]

                @pl.loop(0, BATCH // 16)
                def _(bg):
                    lvec = idx_s[xcol, pl.ds(bg * 16, 16)]
                    for bs in range(16):
                        lvl = lvec[bs]
                        b = bg * 16 + bs
                        for c in range(NC16):
                            plsc.addupdate(
                                acc_s.at[b, pl.ds(c * 16, 16)],
                                vt_s[lvl, pl.ds(c * 16, 16)] * wvec[c])

        one = jnp.full((16,), 1.0, jnp.float32)
        mone = jnp.full((16,), -1.0, jnp.float32)

        @pl.loop(0, BATCH)
        def _(b):
            for c in range(NC16):
                v = acc_s[b, pl.ds(c * 16, 16)]
                acc_s[b, pl.ds(c * 16, 16)] = jnp.where(v > 0.0, one, mone)

        pltpu.sync_copy(acc_s, out_hbm.at[w])

    return enc(img_t, vt_r, xt_r, yt_r)


def kernel(x, value_table, x_table, y_table):
    img = x.reshape(BATCH, SIZE * SIZE).astype(jnp.int32)
    img_t = img.T

    pad = DPAD - DIM
    vt = jnp.pad(value_table, ((0, 0), (0, pad)))
    xt = jnp.pad(x_table, ((0, 0), (0, pad)))
    yt = jnp.pad(y_table, ((0, 0), (0, pad)))
    vt_r = vt.reshape(LEVELS, NW, CPW).transpose(1, 0, 2)
    xt_r = xt.reshape(SIZE, NW, CPW).transpose(1, 0, 2)
    yt_r = yt.reshape(SIZE, NW, CPW).transpose(1, 0, 2)

    out = _sc_encode(img_t, vt_r, xt_r, yt_r)
    return out.transpose(1, 0, 2).reshape(BATCH, DPAD)[:, :DIM]


# speedup vs baseline: 6.4810x; 6.4810x over previous
import functools

import jax
import jax.numpy as jnp
from jax import lax
from jax.experimental import pallas as pl
from jax.experimental.pallas import tpu as pltpu
from jax.experimental.pallas import tpu_sc as plsc

DIM = 10000
SIZE = 32
LEVELS = 256
BATCH = 64

NW = 32
CPW = 320
DPAD = NW * CPW
XG = 16
NXG = SIZE // XG
NCP = 2
CPASS = CPW // (16 * NCP)


def _sc_encode(img_r, vt_r, xt_r, yt_r):
    mesh = plsc.VectorSubcoreMesh(core_axis_name="c", subcore_axis_name="s")

    @functools.partial(
        pl.kernel,
        mesh=mesh,
        out_type=jax.ShapeDtypeStruct((NW, BATCH, CPW), jnp.float32),
        scratch_types=[
            pltpu.VMEM((LEVELS, CPW), jnp.float32),
            pltpu.VMEM((SIZE, CPW), jnp.float32),
            pltpu.VMEM((CPW,), jnp.float32),
            pltpu.VMEM((BATCH, CPW), jnp.float32),
            pltpu.VMEM((BATCH, SIZE), jnp.int32),
            pltpu.SemaphoreType.DMA,
            pltpu.SemaphoreType.DMA,
        ],
        compiler_params=pltpu.CompilerParams(use_tc_tiling_on_sc=False),
    )
    def enc(img_hbm, vt_hbm, xt_hbm, yt_hbm, out_hbm,
            vt_s, xt_s, yt_row, acc_s, idx_s, sem1, sem2):
        w = lax.axis_index("s") * 2 + lax.axis_index("c")
        cp1 = pltpu.async_copy(vt_hbm.at[w], vt_s, sem1)
        cp2 = pltpu.async_copy(xt_hbm.at[w], xt_s, sem2)

        zero = jnp.zeros((16,), jnp.float32)

        @pl.loop(0, BATCH)
        def _(b):
            for c in range(CPW // 16):
                acc_s[b, pl.ds(c * 16, 16)] = zero

        cp1.wait()
        cp2.wait()

        @pl.loop(0, SIZE)
        def _(y):
            pltpu.sync_copy(img_hbm.at[y], idx_s)
            pltpu.sync_copy(yt_hbm.at[w, y], yt_row)

            @pl.loop(0, CPASS)
            def _(cp):
                cbase = cp * (16 * NCP)
                for xg in range(NXG):
                    wv = [[xt_s[xg * XG + p, pl.ds(cbase + c * 16, 16)]
                           * yt_row[pl.ds(cbase + c * 16, 16)]
                           for c in range(NCP)]
                          for p in range(XG)]

                    @plsc.parallel_loop(0, BATCH)
                    def _(b):
                        lvec = idx_s[b, pl.ds(xg * XG, 16)]
                        for c in range(NCP):
                            acc = None
                            for p in range(XG):
                                term = (vt_s[lvec[p],
                                             pl.ds(cbase + c * 16, 16)]
                                        * wv[p][c])
                                acc = term if acc is None else acc + term
                            plsc.addupdate(
                                acc_s.at[b, pl.ds(cbase + c * 16, 16)], acc)

        one = jnp.full((16,), 1.0, jnp.float32)
        mone = jnp.full((16,), -1.0, jnp.float32)

        @pl.loop(0, BATCH)
        def _(b):
            for c in range(CPW // 16):
                v = acc_s[b, pl.ds(c * 16, 16)]
                acc_s[b, pl.ds(c * 16, 16)] = jnp.where(v > 0.0, one, mone)

        pltpu.sync_copy(acc_s, out_hbm.at[w])

    return enc(img_r, vt_r, xt_r, yt_r)


def kernel(x, value_table, x_table, y_table):
    img = x.reshape(BATCH, SIZE, SIZE).astype(jnp.int32)
    img_r = img.transpose(1, 0, 2)

    pad = DPAD - DIM
    vt = jnp.pad(value_table, ((0, 0), (0, pad)))
    xt = jnp.pad(x_table, ((0, 0), (0, pad)))
    yt = jnp.pad(y_table, ((0, 0), (0, pad)))
    vt_r = vt.reshape(LEVELS, NW, CPW).transpose(1, 0, 2)
    xt_r = xt.reshape(SIZE, NW, CPW).transpose(1, 0, 2)
    yt_r = yt.reshape(SIZE, NW, CPW).transpose(1, 0, 2)

    out = _sc_encode(img_r, vt_r, xt_r, yt_r)
    return out.transpose(1, 0, 2).reshape(BATCH, DPAD)[:, :DIM]
